# baseline (speedup 1.0000x reference)
import functools
import math

import jax
import jax.numpy as jnp
from jax import lax
from jax.experimental import pallas as pl
from jax.experimental.pallas import tpu as pltpu
from jax.experimental.pallas import tpu_sc as plsc

D_MODEL = 768
SCALE = math.sqrt(D_MODEL)
NC, NS, LANES = 2, 16, 16
NW = NC * NS
CHUNK = 64
NBUF = 2
SUB = CHUNK // 2


def _scale_rows(buf, start, nrows):
    def row_body(r, carry):
        for c in range(D_MODEL // LANES):
            sl = pl.ds(c * LANES, LANES)
            buf[r, sl] = buf[r, sl] * SCALE
        return carry

    lax.fori_loop(start, start + nrows, row_body, 0)


def _emb_body(nchunks, b_per_w, x_hbm, tab_hbm, out_hbm, idx_v, rows_v, *sems):
    gs, ss = sems[:2 * NBUF], sems[2 * NBUF:]
    wid = lax.axis_index("s") * NC + lax.axis_index("c")
    base = wid * b_per_w
    pltpu.sync_copy(x_hbm.at[wid], idx_v)

    def start_gather(j, b):
        for h in range(2):
            src = tab_hbm.at[idx_v.at[2 * j + h]]
            dst = rows_v.at[b].at[pl.ds(h * SUB, SUB)]
            pltpu.async_copy(src, dst, gs[2 * b + h])

    def wait_gather(b, h):
        dst = rows_v.at[b].at[pl.ds(h * SUB, SUB)]
        pltpu.make_async_copy(tab_hbm.at[idx_v.at[0]], dst, gs[2 * b + h]).wait()

    def start_store(j, b, row0, nrows):
        src = rows_v.at[b].at[pl.ds(row0, nrows)]
        dst = out_hbm.at[pl.ds(base + j * CHUNK + row0, nrows)]
        pltpu.async_copy(src, dst, ss[b])

    def wait_store(b):
        dst = out_hbm.at[pl.ds(base, CHUNK)]
        pltpu.make_async_copy(rows_v.at[b], dst, ss[b]).wait()

    def process(j, b, store_piece):
        for h in range(2):
            wait_gather(b, h)
            _scale_rows(rows_v.at[b], h * SUB, SUB)
            for p in range(SUB // store_piece):
                start_store(j, b, h * SUB + p * store_piece, store_piece)

    for b in range(NBUF):
        start_gather(b, b)

    ngroups = nchunks // NBUF

    def group_body(g, carry):
        for b in range(NBUF):
            process(g * NBUF + b, b, SUB)
        for b in range(NBUF):
            wait_store(b)
            start_gather((g + 1) * NBUF + b, b)
        return carry

    lax.fori_loop(0, ngroups - 1, group_body, 0)

    g = ngroups - 1
    process(g * NBUF, 0, SUB)
    process(g * NBUF + 1, 1, SUB // 2)
    for b in range(NBUF):
        wait_store(b)


def kernel(x, embedding_weight):
    orig_shape = x.shape
    b_total = x.size
    b_per_w = b_total // NW
    nchunks = b_per_w // CHUNK
    x_resh = x.reshape(NW, 2 * nchunks, SUB).astype(jnp.int32)

    mesh = plsc.VectorSubcoreMesh(core_axis_name="c", subcore_axis_name="s")
    emb = pl.kernel(
        functools.partial(_emb_body, nchunks, b_per_w),
        out_type=jax.ShapeDtypeStruct((b_total, D_MODEL), jnp.float32),
        mesh=mesh,
        compiler_params=pltpu.CompilerParams(use_tc_tiling_on_sc=False),
        scratch_types=[
            pltpu.VMEM((2 * nchunks, SUB), jnp.int32),
            pltpu.VMEM((NBUF, CHUNK, D_MODEL), jnp.float32),
        ] + [pltpu.SemaphoreType.DMA] * (3 * NBUF),
    )
    out = emb(x_resh, embedding_weight)
    return out.reshape(orig_shape + (D_MODEL,))


# speedup vs baseline: 6.8563x; 6.8563x over previous
import functools
import math

import jax
import jax.numpy as jnp
from jax import lax
from jax.experimental import pallas as pl
from jax.experimental.pallas import tpu as pltpu
from jax.experimental.pallas import tpu_sc as plsc

D_MODEL = 768
SCALE = math.sqrt(D_MODEL)
NC, NS, LANES = 2, 16, 16
NW = NC * NS
CHUNK = 64
NBUF = 2
SUB = CHUNK // 2


def _scale_rows(buf, start, nrows):
    def row_body(r, carry):
        for c in range(D_MODEL // LANES):
            sl = pl.ds(c * LANES, LANES)
            buf[r, sl] = buf[r, sl] * SCALE
        return carry

    lax.fori_loop(start, start + nrows, row_body, 0)


def _emb_body(nchunks, b_per_w, x_hbm, tab_hbm, out_hbm, idx_v, rows_v, *sems):
    gs, ss = sems[:2 * NBUF], sems[2 * NBUF:]
    wid = lax.axis_index("s") * NC + lax.axis_index("c")
    base = wid * b_per_w
    pltpu.sync_copy(x_hbm.at[wid], idx_v)

    def start_gather(j, b):
        for h in range(2):
            src = tab_hbm.at[idx_v.at[2 * j + h]]
            dst = rows_v.at[b].at[pl.ds(h * SUB, SUB)]
            pltpu.async_copy(src, dst, gs[2 * b + h])

    def wait_gather(b, h):
        dst = rows_v.at[b].at[pl.ds(h * SUB, SUB)]
        pltpu.make_async_copy(tab_hbm.at[idx_v.at[0]], dst, gs[2 * b + h]).wait()

    def start_store(j, b, row0, nrows):
        src = rows_v.at[b].at[pl.ds(row0, nrows)]
        dst = out_hbm.at[pl.ds(base + j * CHUNK + row0, nrows)]
        pltpu.async_copy(src, dst, ss[b])

    def wait_store(b):
        dst = out_hbm.at[pl.ds(base, CHUNK)]
        pltpu.make_async_copy(rows_v.at[b], dst, ss[b]).wait()

    def process(j, b, store_piece):
        for h in range(2):
            wait_gather(b, h)
            _scale_rows(rows_v.at[b], h * SUB, SUB)
            for p in range(SUB // store_piece):
                start_store(j, b, h * SUB + p * store_piece, store_piece)

    for b in range(NBUF):
        start_gather(b, b)

    ngroups = nchunks // NBUF

    def group_body(g, carry):
        for b in range(NBUF):
            process(g * NBUF + b, b, SUB)
            wait_store(b)
            start_gather((g + 1) * NBUF + b, b)
        return carry

    lax.fori_loop(0, ngroups - 1, group_body, 0)

    g = ngroups - 1
    process(g * NBUF, 0, SUB)
    process(g * NBUF + 1, 1, SUB // 2)
    for b in range(NBUF):
        wait_store(b)


def kernel(x, embedding_weight):
    orig_shape = x.shape
    b_total = x.size
    b_per_w = b_total // NW
    nchunks = b_per_w // CHUNK
    x_resh = x.reshape(NW, 2 * nchunks, SUB).astype(jnp.int32)

    mesh = plsc.VectorSubcoreMesh(core_axis_name="c", subcore_axis_name="s")
    emb = pl.kernel(
        functools.partial(_emb_body, nchunks, b_per_w),
        out_type=jax.ShapeDtypeStruct((b_total, D_MODEL), jnp.float32),
        mesh=mesh,
        scratch_types=[
            pltpu.VMEM((2 * nchunks, SUB), jnp.int32),
            pltpu.VMEM((NBUF, CHUNK, D_MODEL), jnp.float32),
        ] + [pltpu.SemaphoreType.DMA] * (3 * NBUF),
    )
    out = emb(x_resh, embedding_weight)
    return out.reshape(orig_shape + (D_MODEL,))


# speedup vs baseline: 6.9949x; 1.0202x over previous
import functools
import math

import jax
import jax.numpy as jnp
from jax import lax
from jax.experimental import pallas as pl
from jax.experimental.pallas import tpu as pltpu
from jax.experimental.pallas import tpu_sc as plsc

D_MODEL = 768
SCALE = math.sqrt(D_MODEL)
NC, NS, LANES = 2, 16, 16
NW = NC * NS
CHUNK = 64
NBUF = 2
SUB = CHUNK // 2


def _scale_rows(buf, start, nrows):
    def row_body(r, carry):
        for c in range(D_MODEL // LANES):
            sl = pl.ds(c * LANES, LANES)
            buf[r, sl] = buf[r, sl] * SCALE
        return carry

    lax.fori_loop(start, start + nrows, row_body, 0)


def _emb_body(nchunks, b_per_w, x_hbm, tab_hbm, out_hbm, idx_v, rows_v, *sems):
    gs, ss = sems[:2 * NBUF], sems[2 * NBUF:]
    wid = lax.axis_index("s") * NC + lax.axis_index("c")
    base = wid * b_per_w
    pltpu.sync_copy(x_hbm.at[wid], idx_v)

    def start_gather_half(j, b, h):
        src = tab_hbm.at[idx_v.at[2 * j + h]]
        dst = rows_v.at[b].at[pl.ds(h * SUB, SUB)]
        pltpu.async_copy(src, dst, gs[2 * b + h])

    def wait_gather(b, h):
        dst = rows_v.at[b].at[pl.ds(h * SUB, SUB)]
        pltpu.make_async_copy(tab_hbm.at[idx_v.at[0]], dst, gs[2 * b + h]).wait()

    def start_store_half(j, b, h):
        src = rows_v.at[b].at[pl.ds(h * SUB, SUB)]
        dst = out_hbm.at[pl.ds(base + j * CHUNK + h * SUB, SUB)]
        pltpu.async_copy(src, dst, ss[2 * b + h])

    def wait_store_half(b, h):
        dst = out_hbm.at[pl.ds(base, SUB)]
        src = rows_v.at[b].at[pl.ds(h * SUB, SUB)]
        pltpu.make_async_copy(src, dst, ss[2 * b + h]).wait()

    def process(j, b):
        for h in range(2):
            wait_gather(b, h)
            _scale_rows(rows_v.at[b], h * SUB, SUB)
            start_store_half(j, b, h)

    for b in range(NBUF):
        for h in range(2):
            start_gather_half(b, b, h)

    ngroups = nchunks // NBUF

    def group_body(g, carry):
        for b in range(NBUF):
            process(g * NBUF + b, b)
            for h in range(2):
                wait_store_half(b, h)
                start_gather_half((g + 1) * NBUF + b, b, h)
        return carry

    lax.fori_loop(0, ngroups - 1, group_body, 0)

    g = ngroups - 1
    for b in range(NBUF):
        process(g * NBUF + b, b)
    for b in range(NBUF):
        for h in range(2):
            wait_store_half(b, h)


def kernel(x, embedding_weight):
    orig_shape = x.shape
    b_total = x.size
    b_per_w = b_total // NW
    nchunks = b_per_w // CHUNK
    x_resh = x.reshape(NW, 2 * nchunks, SUB).astype(jnp.int32)

    mesh = plsc.VectorSubcoreMesh(core_axis_name="c", subcore_axis_name="s")
    emb = pl.kernel(
        functools.partial(_emb_body, nchunks, b_per_w),
        out_type=jax.ShapeDtypeStruct((b_total, D_MODEL), jnp.float32),
        mesh=mesh,
        scratch_types=[
            pltpu.VMEM((2 * nchunks, SUB), jnp.int32),
            pltpu.VMEM((NBUF, CHUNK, D_MODEL), jnp.float32),
        ] + [pltpu.SemaphoreType.DMA] * (4 * NBUF),
    )
    out = emb(x_resh, embedding_weight)
    return out.reshape(orig_shape + (D_MODEL,))
